# baseline (speedup 1.0000x reference)
import functools

import jax
import jax.numpy as jnp
from jax import lax
from jax.experimental import pallas as pl
from jax.experimental.pallas import tpu as pltpu
from jax.experimental.pallas import tpu_sc as plsc

N = 10000
E = 320000
WIDTH = 128
LANES = 16
NF = WIDTH // LANES

NC = 2
NS = 16
NW = NC * NS

NPT = 320
NPAD = NPT * NW

SEG = 2000
NSEG = E // SEG
CH = 64
QCAP = 4096
DRAIN_T = QCAP - SEG - CH


def _sc_segmin(src, dst, x):
    mesh = plsc.VectorSubcoreMesh(
        core_axis_name="c", subcore_axis_name="s",
        num_cores=NC, num_subcores=NS)

    @functools.partial(
        pl.kernel,
        out_type=jax.ShapeDtypeStruct((NPAD, WIDTH), jnp.float32),
        mesh=mesh,
        compiler_params=pltpu.CompilerParams(needs_layout_passes=False),
        scratch_types=[
            pltpu.VMEM((SEG,), jnp.int32),
            pltpu.VMEM((SEG,), jnp.int32),
            pltpu.VMEM((QCAP + LANES + LANES,), jnp.int32),
            pltpu.VMEM((QCAP + LANES + LANES,), jnp.int32),
            pltpu.VMEM((CH + LANES,), jnp.int32),
            pltpu.VMEM((CH + LANES,), jnp.int32),
            pltpu.VMEM((CH, WIDTH), jnp.float32),
            pltpu.VMEM((CH, WIDTH), jnp.float32),
            pltpu.VMEM((NPT, WIDTH), jnp.float32),
            pltpu.SemaphoreType.DMA,
            pltpu.SemaphoreType.DMA,
        ],
    )
    def k(src_hbm, dst_hbm, x_hbm, out_hbm,
          dseg, sseg, qsrc, qdst, gi, gd, rows, rows2, acc, sem, sem2):
        cid = lax.axis_index("c")
        sid = lax.axis_index("s")
        wid = sid * NC + cid
        lo = wid * NPT

        inf16 = jnp.full((LANES,), jnp.inf, jnp.float32)
        zero16 = jnp.zeros((LANES,), jnp.int32)

        def init_acc(r, _):
            for f in range(NF):
                acc[r, pl.ds(f * LANES, LANES)] = inf16
            return 0
        lax.fori_loop(0, NPT, init_acc, 0)

        def zero_q(i, _):
            qsrc[pl.ds(i * LANES, LANES)] = zero16
            qdst[pl.ds(i * LANES, LANES)] = zero16
            return 0
        lax.fori_loop(0, (QCAP + 2 * LANES) // LANES, zero_q, 0)
        for i in range(CH // LANES + 1):
            gi[pl.ds(i * LANES, LANES)] = zero16
            gd[pl.ds(i * LANES, LANES)] = zero16

        def rmw_snap(nb, buf):
            def e_body(i, _):
                d = gd[pl.ds(i, LANES)][0]
                for f in range(NF):
                    sl = pl.ds(f * LANES, LANES)
                    acc[d, sl] = jnp.minimum(acc[d, sl], buf[i, sl])
                return 0
            lax.fori_loop(0, nb, e_body, 0)

        def drain_all(qn):
            def ch_body(c, _):
                pltpu.async_copy(
                    x_hbm.at[qsrc.at[pl.ds(c * CH, CH)]], rows2, sem2).wait()

                def e_body(i, _):
                    d = qdst[pl.ds(c * CH + i, LANES)][0]
                    for f in range(NF):
                        sl = pl.ds(f * LANES, LANES)
                        acc[d, sl] = jnp.minimum(acc[d, sl], rows2[i, sl])
                    return 0
                lax.fori_loop(0, jnp.clip(qn - c * CH, 0, CH), e_body, 0)
                return 0
            lax.fori_loop(0, (qn + CH - 1) // CH, ch_body, 0)

        gdesc = pltpu.async_copy(x_hbm.at[gi.at[pl.ds(0, CH)]], rows, sem)

        def seg_body(s, carry):
            qn, nb_prev = carry
            pltpu.sync_copy(dst_hbm.at[pl.ds(s * SEG, SEG)], dseg)
            pltpu.sync_copy(src_hbm.at[pl.ds(s * SEG, SEG)], sseg)

            def scan_body(j, qn):
                dv = dseg[pl.ds(j * LANES, LANES)]
                sv = sseg[pl.ds(j * LANES, LANES)]
                dl = dv - lo
                m = (dl >= 0) & (dl < NPT)
                pc = plsc.all_reduce_population_count(m)[0]

                @pl.when(pc > 0)
                def _():
                    cum = plsc.cumsum(m.astype(jnp.int32))
                    pos = jnp.where(m, qn + cum - 1, QCAP)
                    plsc.store_scatter(qsrc, [pos], sv)
                    plsc.store_scatter(qdst, [pos], dl)
                return qn + pc
            qn = lax.fori_loop(0, SEG // LANES, scan_body, qn)

            pltpu.make_async_copy(x_hbm.at[gi.at[pl.ds(0, CH)]], rows, sem).wait()
            rmw_snap(nb_prev, rows)

            @pl.when(qn > DRAIN_T)
            def _():
                drain_all(qn)
            qn = jnp.where(qn > DRAIN_T, 0, qn)

            for i in range(CH // LANES):
                gi[pl.ds(i * LANES, LANES)] = qsrc[pl.ds(i * LANES, LANES)]
                gd[pl.ds(i * LANES, LANES)] = qdst[pl.ds(i * LANES, LANES)]
            pltpu.async_copy(x_hbm.at[gi.at[pl.ds(0, CH)]], rows, sem)

            nb = jnp.minimum(qn, CH)
            rem = qn - nb

            def mv(i, _):
                v1 = qsrc[pl.ds(CH + i * LANES, LANES)]
                qsrc[pl.ds(i * LANES, LANES)] = v1
                v2 = qdst[pl.ds(CH + i * LANES, LANES)]
                qdst[pl.ds(i * LANES, LANES)] = v2
                return 0
            lax.fori_loop(0, (rem + LANES - 1) // LANES, mv, 0)
            return (rem, nb)
        qn, nb_last = lax.fori_loop(
            0, NSEG, seg_body, (jnp.int32(0), jnp.int32(0)))
        pltpu.make_async_copy(x_hbm.at[gi.at[pl.ds(0, CH)]], rows, sem).wait()
        rmw_snap(nb_last, rows)
        drain_all(qn)

        pltpu.sync_copy(acc, out_hbm.at[pl.ds(wid * NPT, NPT)])

    return k(src, dst, x)


ROWS_BLK = 1000


def _tc_epilogue(x, segmin, w1, w2, b):
    def body(x_ref, s_ref, w1_ref, w2_ref, b_ref, o_ref):
        xb = x_ref[...]
        sb = s_ref[...]
        maxes = jnp.where(sb == jnp.inf, 0.0, xb - sb)
        h = jnp.dot(xb, w1_ref[...], preferred_element_type=jnp.float32)
        h = h + jnp.dot(maxes, w2_ref[...], preferred_element_type=jnp.float32)
        h = h + b_ref[...]
        o_ref[...] = xb + jnp.maximum(h, 0.0)

    grid = (N // ROWS_BLK,)
    return pl.pallas_call(
        body,
        grid=grid,
        in_specs=[
            pl.BlockSpec((ROWS_BLK, WIDTH), lambda i: (i, 0)),
            pl.BlockSpec((ROWS_BLK, WIDTH), lambda i: (i, 0)),
            pl.BlockSpec((WIDTH, WIDTH), lambda i: (0, 0)),
            pl.BlockSpec((WIDTH, WIDTH), lambda i: (0, 0)),
            pl.BlockSpec((1, WIDTH), lambda i: (0, 0)),
        ],
        out_specs=pl.BlockSpec((ROWS_BLK, WIDTH), lambda i: (i, 0)),
        out_shape=jax.ShapeDtypeStruct((N, WIDTH), jnp.float32),
    )(x, segmin, w1, w2, b)


def kernel(x, e, W, b):
    src = e[0]
    dst = e[1]
    segmin = _sc_segmin(src, dst, x)[:N]
    w1 = W[:WIDTH]
    w2 = W[WIDTH:]
    return _tc_epilogue(x, segmin, w1, w2, b.reshape(1, WIDTH))


# speedup vs baseline: 1.2007x; 1.2007x over previous
import functools

import jax
import jax.numpy as jnp
from jax import lax
from jax.experimental import pallas as pl
from jax.experimental.pallas import tpu as pltpu
from jax.experimental.pallas import tpu_sc as plsc

N = 10000
E = 320000
WIDTH = 128
LANES = 16
NF = WIDTH // LANES

NC = 2
NS = 16
NW = NC * NS

NPT = 320
NPAD = NPT * NW

SEG = 2000
NSEG = E // SEG
CH = 64
QCAP = 4096
DRAIN_T = QCAP - SEG - CH


def _sc_segmin(src, dst, x):
    mesh = plsc.VectorSubcoreMesh(
        core_axis_name="c", subcore_axis_name="s",
        num_cores=NC, num_subcores=NS)

    @functools.partial(
        pl.kernel,
        out_type=jax.ShapeDtypeStruct((NPAD, WIDTH), jnp.float32),
        mesh=mesh,
        compiler_params=pltpu.CompilerParams(needs_layout_passes=False),
        scratch_types=[
            pltpu.VMEM((SEG,), jnp.int32),
            pltpu.VMEM((SEG,), jnp.int32),
            pltpu.VMEM((QCAP + LANES + LANES,), jnp.int32),
            pltpu.VMEM((QCAP + LANES + LANES,), jnp.int32),
            pltpu.VMEM((CH + LANES,), jnp.int32),
            pltpu.VMEM((CH + LANES,), jnp.int32),
            pltpu.VMEM((CH, WIDTH), jnp.float32),
            pltpu.VMEM((CH, WIDTH), jnp.float32),
            pltpu.VMEM((NPT, WIDTH), jnp.float32),
            pltpu.SemaphoreType.DMA,
            pltpu.SemaphoreType.DMA,
        ],
    )
    def k(src_hbm, dst_hbm, x_hbm, out_hbm,
          dseg, sseg, qsrc, qdst, gi, gd, rows, rows2, acc, sem, sem2):
        cid = lax.axis_index("c")
        sid = lax.axis_index("s")
        wid = sid * NC + cid
        lo = wid * NPT

        inf16 = jnp.full((LANES,), jnp.inf, jnp.float32)
        zero16 = jnp.zeros((LANES,), jnp.int32)

        def init_acc(r, _):
            for f in range(NF):
                acc[r, pl.ds(f * LANES, LANES)] = inf16
            return 0
        lax.fori_loop(0, NPT, init_acc, 0)

        def zero_q(i, _):
            qsrc[pl.ds(i * LANES, LANES)] = zero16
            qdst[pl.ds(i * LANES, LANES)] = zero16
            return 0
        lax.fori_loop(0, (QCAP + 2 * LANES) // LANES, zero_q, 0)
        for i in range(CH // LANES + 1):
            gi[pl.ds(i * LANES, LANES)] = zero16
            gd[pl.ds(i * LANES, LANES)] = zero16

        def rmw_snap(nb, buf):
            def e_body(i, _):
                d = gd[pl.ds(i, LANES)][0]
                for f in range(NF):
                    sl = pl.ds(f * LANES, LANES)
                    acc[d, sl] = jnp.minimum(acc[d, sl], buf[i, sl])
                return 0
            lax.fori_loop(0, nb, e_body, 0)

        def drain_all(qn):
            def ch_body(c, _):
                pltpu.async_copy(
                    x_hbm.at[qsrc.at[pl.ds(c * CH, CH)]], rows2, sem2).wait()

                def e_body(i, _):
                    d = qdst[pl.ds(c * CH + i, LANES)][0]
                    for f in range(NF):
                        sl = pl.ds(f * LANES, LANES)
                        acc[d, sl] = jnp.minimum(acc[d, sl], rows2[i, sl])
                    return 0
                lax.fori_loop(0, jnp.clip(qn - c * CH, 0, CH), e_body, 0)
                return 0
            lax.fori_loop(0, (qn + CH - 1) // CH, ch_body, 0)

        gdesc = pltpu.async_copy(x_hbm.at[gi.at[pl.ds(0, CH)]], rows, sem)

        def seg_body(s, carry):
            qn, nb_prev = carry
            pltpu.sync_copy(dst_hbm.at[pl.ds(s * SEG, SEG)], dseg)
            pltpu.sync_copy(src_hbm.at[pl.ds(s * SEG, SEG)], sseg)

            def scan_body(j, qn):
                dv = dseg[pl.ds(j * LANES, LANES)]
                sv = sseg[pl.ds(j * LANES, LANES)]
                dl = dv - lo
                m = (dl >= 0) & (dl < NPT)
                cum = plsc.cumsum(m.astype(jnp.int32))
                pos = jnp.where(m, qn + cum - 1, QCAP)
                plsc.store_scatter(qsrc, [pos], sv)
                plsc.store_scatter(qdst, [pos], dl)
                return qn + cum[LANES - 1]
            qn = lax.fori_loop(0, SEG // LANES, scan_body, qn)

            pltpu.make_async_copy(x_hbm.at[gi.at[pl.ds(0, CH)]], rows, sem).wait()
            rmw_snap(nb_prev, rows)

            @pl.when(qn > DRAIN_T)
            def _():
                drain_all(qn)
            qn = jnp.where(qn > DRAIN_T, 0, qn)

            for i in range(CH // LANES):
                gi[pl.ds(i * LANES, LANES)] = qsrc[pl.ds(i * LANES, LANES)]
                gd[pl.ds(i * LANES, LANES)] = qdst[pl.ds(i * LANES, LANES)]
            pltpu.async_copy(x_hbm.at[gi.at[pl.ds(0, CH)]], rows, sem)

            nb = jnp.minimum(qn, CH)
            rem = qn - nb

            def mv(i, _):
                v1 = qsrc[pl.ds(CH + i * LANES, LANES)]
                qsrc[pl.ds(i * LANES, LANES)] = v1
                v2 = qdst[pl.ds(CH + i * LANES, LANES)]
                qdst[pl.ds(i * LANES, LANES)] = v2
                return 0
            lax.fori_loop(0, (rem + LANES - 1) // LANES, mv, 0)
            return (rem, nb)
        qn, nb_last = lax.fori_loop(
            0, NSEG, seg_body, (jnp.int32(0), jnp.int32(0)))
        pltpu.make_async_copy(x_hbm.at[gi.at[pl.ds(0, CH)]], rows, sem).wait()
        rmw_snap(nb_last, rows)
        drain_all(qn)

        pltpu.sync_copy(acc, out_hbm.at[pl.ds(wid * NPT, NPT)])

    return k(src, dst, x)


ROWS_BLK = 1000


def _tc_epilogue(x, segmin, w1, w2, b):
    def body(x_ref, s_ref, w1_ref, w2_ref, b_ref, o_ref):
        xb = x_ref[...]
        sb = s_ref[...]
        maxes = jnp.where(sb == jnp.inf, 0.0, xb - sb)
        h = jnp.dot(xb, w1_ref[...], preferred_element_type=jnp.float32)
        h = h + jnp.dot(maxes, w2_ref[...], preferred_element_type=jnp.float32)
        h = h + b_ref[...]
        o_ref[...] = xb + jnp.maximum(h, 0.0)

    grid = (N // ROWS_BLK,)
    return pl.pallas_call(
        body,
        grid=grid,
        in_specs=[
            pl.BlockSpec((ROWS_BLK, WIDTH), lambda i: (i, 0)),
            pl.BlockSpec((ROWS_BLK, WIDTH), lambda i: (i, 0)),
            pl.BlockSpec((WIDTH, WIDTH), lambda i: (0, 0)),
            pl.BlockSpec((WIDTH, WIDTH), lambda i: (0, 0)),
            pl.BlockSpec((1, WIDTH), lambda i: (0, 0)),
        ],
        out_specs=pl.BlockSpec((ROWS_BLK, WIDTH), lambda i: (i, 0)),
        out_shape=jax.ShapeDtypeStruct((N, WIDTH), jnp.float32),
    )(x, segmin, w1, w2, b)


def kernel(x, e, W, b):
    src = e[0]
    dst = e[1]
    segmin = _sc_segmin(src, dst, x)[:N]
    w1 = W[:WIDTH]
    w2 = W[WIDTH:]
    return _tc_epilogue(x, segmin, w1, w2, b.reshape(1, WIDTH))


# speedup vs baseline: 1.3007x; 1.0833x over previous
import functools

import jax
import jax.numpy as jnp
from jax import lax
from jax.experimental import pallas as pl
from jax.experimental.pallas import tpu as pltpu
from jax.experimental.pallas import tpu_sc as plsc

N = 10000
E = 320000
WIDTH = 128
LANES = 16
NF = WIDTH // LANES

NC = 2
NS = 16
NW = NC * NS

NPT = 320
NPAD = NPT * NW

SEG = 2000
NSEG = E // SEG
CH = 64
QCAP = 4096
DRAIN_T = QCAP - SEG - CH


def _sc_segmin(src, dst, x):
    mesh = plsc.VectorSubcoreMesh(
        core_axis_name="c", subcore_axis_name="s",
        num_cores=NC, num_subcores=NS)

    @functools.partial(
        pl.kernel,
        out_type=jax.ShapeDtypeStruct((NPAD, WIDTH), jnp.float32),
        mesh=mesh,
        compiler_params=pltpu.CompilerParams(needs_layout_passes=False),
        scratch_types=[
            pltpu.VMEM((SEG,), jnp.int32),
            pltpu.VMEM((SEG,), jnp.int32),
            pltpu.VMEM((QCAP + LANES + LANES,), jnp.int32),
            pltpu.VMEM((QCAP + LANES + LANES,), jnp.int32),
            pltpu.VMEM((CH + LANES,), jnp.int32),
            pltpu.VMEM((CH + LANES,), jnp.int32),
            pltpu.VMEM((CH, WIDTH), jnp.float32),
            pltpu.VMEM((CH, WIDTH), jnp.float32),
            pltpu.VMEM((NPT, WIDTH), jnp.float32),
            pltpu.SemaphoreType.DMA,
            pltpu.SemaphoreType.DMA,
            pltpu.SemaphoreType.DMA,
        ],
    )
    def k(src_hbm, dst_hbm, x_hbm, out_hbm,
          dseg, sseg, qsrc, qdst, gi, gd, rows, rows2, acc, sem, sem2, semi):
        cid = lax.axis_index("c")
        sid = lax.axis_index("s")
        wid = sid * NC + cid
        lo = wid * NPT

        inf16 = jnp.full((LANES,), jnp.inf, jnp.float32)
        zero16 = jnp.zeros((LANES,), jnp.int32)

        def init_acc(r, _):
            for f in range(NF):
                acc[r, pl.ds(f * LANES, LANES)] = inf16
            return 0
        lax.fori_loop(0, NPT, init_acc, 0)

        def zero_q(i, _):
            qsrc[pl.ds(i * LANES, LANES)] = zero16
            qdst[pl.ds(i * LANES, LANES)] = zero16
            return 0
        lax.fori_loop(0, (QCAP + 2 * LANES) // LANES, zero_q, 0)
        for i in range(CH // LANES + 1):
            gi[pl.ds(i * LANES, LANES)] = zero16
            gd[pl.ds(i * LANES, LANES)] = zero16

        def rmw_snap(nb, buf):
            def e_body(i, _):
                d = gd[pl.ds(i, LANES)][0]
                for f in range(NF):
                    sl = pl.ds(f * LANES, LANES)
                    acc[d, sl] = jnp.minimum(acc[d, sl], buf[i, sl])
                return 0
            lax.fori_loop(0, nb, e_body, 0)

        def drain_all(qn):
            def ch_body(c, _):
                pltpu.async_copy(
                    x_hbm.at[qsrc.at[pl.ds(c * CH, CH)]], rows2, sem2).wait()

                def e_body(i, _):
                    d = qdst[pl.ds(c * CH + i, LANES)][0]
                    for f in range(NF):
                        sl = pl.ds(f * LANES, LANES)
                        acc[d, sl] = jnp.minimum(acc[d, sl], rows2[i, sl])
                    return 0
                lax.fori_loop(0, jnp.clip(qn - c * CH, 0, CH), e_body, 0)
                return 0
            lax.fori_loop(0, (qn + CH - 1) // CH, ch_body, 0)

        pltpu.async_copy(x_hbm.at[gi.at[pl.ds(0, CH)]], rows, sem)
        pltpu.async_copy(dst_hbm.at[pl.ds(0, SEG)], dseg, semi)
        pltpu.async_copy(src_hbm.at[pl.ds(0, SEG)], sseg, semi)

        def seg_body(s, carry):
            qn, nb_prev = carry
            pltpu.make_async_copy(dst_hbm.at[pl.ds(0, SEG)], dseg, semi).wait()
            pltpu.make_async_copy(src_hbm.at[pl.ds(0, SEG)], sseg, semi).wait()

            def scan_body(j, qn):
                dv = dseg[pl.ds(j * LANES, LANES)]
                sv = sseg[pl.ds(j * LANES, LANES)]
                dl = dv - lo
                m = (dl >= 0) & (dl < NPT)
                cum = plsc.cumsum(m.astype(jnp.int32))
                pos = jnp.where(m, qn + cum - 1, QCAP)
                plsc.store_scatter(qsrc, [pos], sv)
                plsc.store_scatter(qdst, [pos], dl)
                return qn + cum[LANES - 1]
            qn = lax.fori_loop(0, SEG // LANES, scan_body, qn)

            pltpu.make_async_copy(x_hbm.at[gi.at[pl.ds(0, CH)]], rows, sem).wait()
            rmw_snap(nb_prev, rows)

            @pl.when(qn > DRAIN_T)
            def _():
                drain_all(qn)
            qn = jnp.where(qn > DRAIN_T, 0, qn)

            for i in range(CH // LANES):
                gi[pl.ds(i * LANES, LANES)] = qsrc[pl.ds(i * LANES, LANES)]
                gd[pl.ds(i * LANES, LANES)] = qdst[pl.ds(i * LANES, LANES)]
            pltpu.async_copy(x_hbm.at[gi.at[pl.ds(0, CH)]], rows, sem)

            nxt = jnp.where(s + 1 < NSEG, s + 1, 0)
            pltpu.async_copy(dst_hbm.at[pl.ds(nxt * SEG, SEG)], dseg, semi)
            pltpu.async_copy(src_hbm.at[pl.ds(nxt * SEG, SEG)], sseg, semi)

            nb = jnp.minimum(qn, CH)
            rem = qn - nb

            def mv(i, _):
                v1 = qsrc[pl.ds(CH + i * LANES, LANES)]
                qsrc[pl.ds(i * LANES, LANES)] = v1
                v2 = qdst[pl.ds(CH + i * LANES, LANES)]
                qdst[pl.ds(i * LANES, LANES)] = v2
                return 0
            lax.fori_loop(0, (rem + LANES - 1) // LANES, mv, 0)
            return (rem, nb)
        qn, nb_last = lax.fori_loop(
            0, NSEG, seg_body, (jnp.int32(0), jnp.int32(0)))
        pltpu.make_async_copy(dst_hbm.at[pl.ds(0, SEG)], dseg, semi).wait()
        pltpu.make_async_copy(src_hbm.at[pl.ds(0, SEG)], sseg, semi).wait()
        pltpu.make_async_copy(x_hbm.at[gi.at[pl.ds(0, CH)]], rows, sem).wait()
        rmw_snap(nb_last, rows)
        drain_all(qn)

        pltpu.sync_copy(acc, out_hbm.at[pl.ds(wid * NPT, NPT)])

    return k(src, dst, x)


ROWS_BLK = 1000


def _tc_epilogue(x, segmin, w1, w2, b):
    def body(x_ref, s_ref, w1_ref, w2_ref, b_ref, o_ref):
        xb = x_ref[...]
        sb = s_ref[...]
        maxes = jnp.where(sb == jnp.inf, 0.0, xb - sb)
        h = jnp.dot(xb, w1_ref[...], preferred_element_type=jnp.float32)
        h = h + jnp.dot(maxes, w2_ref[...], preferred_element_type=jnp.float32)
        h = h + b_ref[...]
        o_ref[...] = xb + jnp.maximum(h, 0.0)

    grid = (N // ROWS_BLK,)
    return pl.pallas_call(
        body,
        grid=grid,
        in_specs=[
            pl.BlockSpec((ROWS_BLK, WIDTH), lambda i: (i, 0)),
            pl.BlockSpec((ROWS_BLK, WIDTH), lambda i: (i, 0)),
            pl.BlockSpec((WIDTH, WIDTH), lambda i: (0, 0)),
            pl.BlockSpec((WIDTH, WIDTH), lambda i: (0, 0)),
            pl.BlockSpec((1, WIDTH), lambda i: (0, 0)),
        ],
        out_specs=pl.BlockSpec((ROWS_BLK, WIDTH), lambda i: (i, 0)),
        out_shape=jax.ShapeDtypeStruct((N, WIDTH), jnp.float32),
    )(x, segmin, w1, w2, b)


def kernel(x, e, W, b):
    src = e[0]
    dst = e[1]
    segmin = _sc_segmin(src, dst, x)[:N]
    w1 = W[:WIDTH]
    w2 = W[WIDTH:]
    return _tc_epilogue(x, segmin, w1, w2, b.reshape(1, WIDTH))
